# baseline (speedup 1.0000x reference)
import functools

import jax
import jax.numpy as jnp
from jax import lax
from jax.experimental import pallas as pl
from jax.experimental.pallas import tpu as pltpu
from jax.experimental.pallas import tpu_sc as plsc

N_NODES = 10000
N_EDGES = 320000
D = 128
DH = D // 2

NS = 16

K = 64
EB = 160
NSUB = 2
E_PAD = NS * NSUB * EB * K
PAD_COL = N_NODES + 200

N_PAD = 10240
DEG_SL = N_PAD // NS

NG = 8
NSR = 3


def _rsqrt16(d):
    i = lax.bitcast_convert_type(d, jnp.int32)
    magic = jnp.full((16,), 0x5F3759DF, jnp.int32)
    y = lax.bitcast_convert_type(magic - lax.shift_right_logical(i, 1), jnp.float32)
    for _ in range(3):
        y = y * (1.5 - 0.5 * d * y * y)
    return jnp.where(d > 0.5, y, 0.0)


def _sc_body(h2_hbm, row_hbm, col_hbm, cns_hbm, out_hbm,
             rowB, colB, cnsB, disv, ov,
             gb0, gb1, gb2, gb3, gb4, gb5, gb6, gb7, sb0, sb1, sb2,
             deg_sh, out_sh, esem,
             gm0, gm1, gm2, gm3, gm4, gm5, gm6, gm7, sm0, sm1, sm2):
    c = lax.axis_index("c")
    s = lax.axis_index("s")
    gbufs = (gb0, gb1, gb2, gb3, gb4, gb5, gb6, gb7)
    sbufs = (sb0, sb1, sb2)
    gsems = (gm0, gm1, gm2, gm3, gm4, gm5, gm6, gm7)
    ssems = (sm0, sm1, sm2)

    @pl.when(c == 0)
    def _core0():
        one16 = jnp.ones((16,), jnp.float32)
        for j in range(K // 16):
            ov[pl.ds(j * 16, 16)] = one16

        def _dz(i, _):
            disv[pl.ds(i * 16, 16)] = jnp.zeros((16,), jnp.float32)
            return _
        lax.fori_loop(0, DEG_SL // 16, _dz, None)

        def _gb_zero(e, _):
            for j in range(DH // 16):
                gb0[e, pl.ds(j * 16, 16)] = jnp.zeros((16,), jnp.float32)
            return _
        lax.fori_loop(0, K, _gb_zero, None)

        obase = s * DEG_SL

        def _zero_out_sh():
            def _fire(q, _):
                pltpu.async_copy(gb0, out_sh.at[pl.ds(obase + q * K, K)], esem)
                return _
            lax.fori_loop(0, DEG_SL // K, _fire, None)

            def _drain(q, _):
                pltpu.make_async_copy(gb0, out_sh.at[pl.ds(obase, K)], esem).wait()
                return _
            lax.fori_loop(0, DEG_SL // K, _drain, None)

        with jax.named_scope("zero"):
            pltpu.sync_copy(disv.at[pl.ds(0, DEG_SL)],
                            deg_sh.at[pl.ds(s * DEG_SL, DEG_SL)])
            _zero_out_sh()
            plsc.subcore_barrier()

        with jax.named_scope("deg"):
            for r in range(NSUB):
                pltpu.sync_copy(col_hbm.at[pl.ds((NSUB * s + r) * EB, EB)], rowB)

                def _deg_fire(i, _):
                    pltpu.async_copy(ov, deg_sh.at[rowB.at[i]], esem, add=True)
                    return _
                lax.fori_loop(0, EB, _deg_fire, None)

                def _deg_drain(i, _):
                    pltpu.make_async_copy(ov, deg_sh.at[rowB.at[0]], esem).wait()
                    return _
                lax.fori_loop(0, EB, _deg_drain, None)
        plsc.subcore_barrier()

        with jax.named_scope("dis"):
            doff = s * DEG_SL
            pltpu.sync_copy(deg_sh.at[pl.ds(doff, DEG_SL)],
                            disv.at[pl.ds(0, DEG_SL)])

            def _dis(i, _):
                sl = pl.ds(i * 16, 16)
                disv[sl] = _rsqrt16(disv[sl])
                return _
            lax.fori_loop(0, DEG_SL // 16, _dis, None)
            pltpu.sync_copy(disv.at[pl.ds(0, DEG_SL)],
                            deg_sh.at[pl.ds(doff, DEG_SL)])
            plsc.subcore_barrier()
            pltpu.sync_copy(deg_sh, disv)

        def _combo(i, _):
            sub = i // 2
            hf = i - sub * 2
            srow = (NSUB * s + sub) * EB

            @pl.when(hf == 0)
            def _():
                with jax.named_scope("slab"):
                    pltpu.sync_copy(row_hbm.at[pl.ds(srow, EB)], rowB)
                    pltpu.sync_copy(col_hbm.at[pl.ds(srow, EB)], colB)
                    pltpu.sync_copy(cns_hbm.at[pl.ds(srow, EB)], cnsB)

                    def _norm(ii, _2):
                        for g in range(K // 16):
                            sl = pl.ds(g * 16, 16)
                            dr = plsc.load_gather(disv, [rowB[ii, sl]])
                            dc = plsc.load_gather(disv, [colB[ii, sl]])
                            cnsB[ii, sl] = dr * dc * jnp.exp(cnsB[ii, sl])
                        return _2
                    lax.fori_loop(0, EB, _norm, None)

            @pl.when(hf == 1)
            def _():
                def _shift(ii, _2):
                    for g in range(K // 16):
                        sl = pl.ds(g * 16, 16)
                        rowB[ii, sl] = rowB[ii, sl] + N_NODES
                    return _2
                lax.fori_loop(0, EB, _shift, None)

            with jax.named_scope("pipe"):
                def _sbuf_zero(e, _2):
                    for j in range(DH // 16):
                        z = jnp.zeros((16,), jnp.float32)
                        sb0[e, pl.ds(j * 16, 16)] = z
                        sb1[e, pl.ds(j * 16, 16)] = z
                        sb2[e, pl.ds(j * 16, 16)] = z
                    return _2
                lax.fori_loop(0, K, _sbuf_zero, None)
                for t in range(NSR):
                    pltpu.async_copy(sbufs[t], out_sh.at[colB.at[0]], ssems[t],
                                     add=True)
                for t in range(NG):
                    pltpu.async_copy(h2_hbm.at[rowB.at[t]], gbufs[t], gsems[t])

                def _oct(q, _2):
                    for t in range(NG):
                        b = NG * q + t
                        st = t % NSR
                        pltpu.make_async_copy(h2_hbm.at[rowB.at[0]], gbufs[t],
                                              gsems[t]).wait()
                        pltpu.make_async_copy(sbufs[st], out_sh.at[colB.at[0]],
                                              ssems[st]).wait()
                        gbuf, sbuf = gbufs[t], sbufs[st]
                        for g in range(K // 16):
                            nv = cnsB[b, pl.ds(g * 16, 16)]
                            r0 = g * 16
                            for e in range(16):
                                sc = nv[e]
                                for jj in range(DH // 16):
                                    sl = pl.ds(jj * 16, 16)
                                    sbuf[r0 + e, sl] = gbuf[r0 + e, sl] * sc
                        pltpu.async_copy(sbufs[st], out_sh.at[colB.at[b]],
                                         ssems[st], add=True)

                        @pl.when(b + NG < EB)
                        def _():
                            pltpu.async_copy(h2_hbm.at[rowB.at[b + NG]],
                                             gbufs[t], gsems[t])
                    return _2
                lax.fori_loop(0, EB // NG, _oct, None)

                for t in range(NSR):
                    pltpu.make_async_copy(sbufs[t], out_sh.at[colB.at[0]],
                                          ssems[t]).wait()
                plsc.subcore_barrier()

            with jax.named_scope("dump"):
                pltpu.sync_copy(out_sh.at[pl.ds(obase, DEG_SL)],
                                out_hbm.at[pl.ds(i * N_PAD + obase, DEG_SL)])

                def _gb_rezero(e, _2):
                    for j in range(DH // 16):
                        gb0[e, pl.ds(j * 16, 16)] = jnp.zeros((16,), jnp.float32)
                    return _2
                lax.fori_loop(0, K, _gb_rezero, None)
                _zero_out_sh()
                plsc.subcore_barrier()
            return _
        lax.fori_loop(0, NSUB * 2, _combo, None)


_sc_propagate = functools.partial(
    pl.kernel,
    out_type=jax.ShapeDtypeStruct((NSUB * 2 * N_PAD, DH), jnp.float32),
    mesh=plsc.VectorSubcoreMesh(core_axis_name="c", subcore_axis_name="s"),
    compiler_params=pltpu.CompilerParams(needs_layout_passes=False,
                                         use_tc_tiling_on_sc=False),
    scratch_types=(
        [
            pltpu.VMEM((EB, K), jnp.int32),
            pltpu.VMEM((EB, K), jnp.int32),
            pltpu.VMEM((EB, K), jnp.float32),
            pltpu.VMEM((N_PAD,), jnp.float32),
            pltpu.VMEM((K,), jnp.float32),
        ]
        + [pltpu.VMEM((K, DH), jnp.float32)] * (NG + NSR)
        + [
            pltpu.VMEM_SHARED((N_PAD,), jnp.float32),
            pltpu.VMEM_SHARED((N_PAD, DH), jnp.float32),
        ]
        + [pltpu.SemaphoreType.DMA] * (1 + NG + NSR)
    ),
)(_sc_body)


def _mm_body(x_ref, w_ref, o_ref):
    o_ref[...] = lax.dot_general(
        x_ref[...], w_ref[...], (((1,), (1,)), ((), ())),
        preferred_element_type=jnp.float32)


def _comb_body(p0, p1, p2, p3, b_ref, o_ref):
    o_ref[:, :DH] = p0[0] + p2[0] + b_ref[0, :DH]
    o_ref[:, DH:] = p1[0] + p3[0] + b_ref[0, DH:]


def kernel(x, edge_index, cns, W, bias):
    n, d_in = x.shape
    d_out = W.shape[0]
    nblk = 10
    h = pl.pallas_call(
        _mm_body,
        grid=(nblk,),
        in_specs=[
            pl.BlockSpec((n // nblk, d_in), lambda i: (i, 0)),
            pl.BlockSpec((d_out, d_in), lambda i: (0, 0)),
        ],
        out_specs=pl.BlockSpec((n // nblk, d_out), lambda i: (i, 0)),
        out_shape=jax.ShapeDtypeStruct((n, d_out), jnp.float32),
    )(x, W)

    n_edges = edge_index.shape[1]
    pad = E_PAD - n_edges
    row2 = jnp.concatenate(
        [edge_index[0], jnp.zeros((pad,), jnp.int32)]).reshape(-1, K)
    pad_cols = N_NODES + (jnp.arange(pad, dtype=jnp.int32) % (N_PAD - N_NODES))
    col2 = jnp.concatenate([edge_index[1], pad_cols]).reshape(-1, K)
    cns2 = jnp.concatenate(
        [cns, jnp.full((pad,), -1e4, cns.dtype)]).reshape(-1, K)
    h2 = jnp.concatenate([h[:, :DH], h[:, DH:]], axis=0)
    part = _sc_propagate(h2, row2, col2, cns2).reshape(NSUB * 2, N_PAD, DH)

    out = pl.pallas_call(
        _comb_body,
        grid=(nblk,),
        in_specs=[
            pl.BlockSpec((1, n // nblk, DH), lambda i: (0, i, 0)),
            pl.BlockSpec((1, n // nblk, DH), lambda i: (1, i, 0)),
            pl.BlockSpec((1, n // nblk, DH), lambda i: (2, i, 0)),
            pl.BlockSpec((1, n // nblk, DH), lambda i: (3, i, 0)),
            pl.BlockSpec((1, d_out), lambda i: (0, 0)),
        ],
        out_specs=pl.BlockSpec((n // nblk, d_out), lambda i: (i, 0)),
        out_shape=jax.ShapeDtypeStruct((n, d_out), jnp.float32),
    )(part, part, part, part, bias.reshape(1, d_out))
    return out


# speedup vs baseline: 1.2343x; 1.2343x over previous
import functools

import jax
import jax.numpy as jnp
from jax import lax
from jax.experimental import pallas as pl
from jax.experimental.pallas import tpu as pltpu
from jax.experimental.pallas import tpu_sc as plsc

N_NODES = 10000
N_EDGES = 320000
D = 128
DH = D // 2

NS = 16

K = 64
EB = 160
NSUB = 2
E_PAD = NS * NSUB * EB * K
PAD_COL = N_NODES + 200

N_PAD = 10240
DEG_SL = N_PAD // NS

NG = 8
NSR = 3


def _rsqrt16(d):
    i = lax.bitcast_convert_type(d, jnp.int32)
    magic = jnp.full((16,), 0x5F3759DF, jnp.int32)
    y = lax.bitcast_convert_type(magic - lax.shift_right_logical(i, 1), jnp.float32)
    for _ in range(3):
        y = y * (1.5 - 0.5 * d * y * y)
    return jnp.where(d > 0.5, y, 0.0)


def _sc_body(h2_hbm, row_hbm, col_hbm, cns_hbm, out_hbm,
             rowB, colB, cnsB, disv, ov,
             gb0, gb1, gb2, gb3, gb4, gb5, gb6, gb7, sb0, sb1, sb2,
             deg_sh, out_sh, esem,
             gm0, gm1, gm2, gm3, gm4, gm5, gm6, gm7, sm0, sm1, sm2):
    c = lax.axis_index("c")
    s = lax.axis_index("s")
    gbufs = (gb0, gb1, gb2, gb3, gb4, gb5, gb6, gb7)
    sbufs = (sb0, sb1, sb2)
    gsems = (gm0, gm1, gm2, gm3, gm4, gm5, gm6, gm7)
    ssems = (sm0, sm1, sm2)

    def _core0():
        one16 = jnp.ones((16,), jnp.float32)
        for j in range(K // 16):
            ov[pl.ds(j * 16, 16)] = one16

        def _dz(i, _):
            disv[pl.ds(i * 16, 16)] = jnp.zeros((16,), jnp.float32)
            return _
        lax.fori_loop(0, DEG_SL // 16, _dz, None)

        def _gb_zero(e, _):
            for j in range(DH // 16):
                gb0[e, pl.ds(j * 16, 16)] = jnp.zeros((16,), jnp.float32)
            return _
        lax.fori_loop(0, K, _gb_zero, None)

        obase = s * DEG_SL

        def _zero_out_sh():
            def _fire(q, _):
                pltpu.async_copy(gb0, out_sh.at[pl.ds(obase + q * K, K)], esem)
                return _
            lax.fori_loop(0, DEG_SL // K, _fire, None)

            def _drain(q, _):
                pltpu.make_async_copy(gb0, out_sh.at[pl.ds(obase, K)], esem).wait()
                return _
            lax.fori_loop(0, DEG_SL // K, _drain, None)

        with jax.named_scope("zero"):
            pltpu.sync_copy(disv.at[pl.ds(0, DEG_SL)],
                            deg_sh.at[pl.ds(s * DEG_SL, DEG_SL)])
            _zero_out_sh()
            plsc.subcore_barrier()

        with jax.named_scope("deg"):
            for r in range(NSUB):
                pltpu.sync_copy(col_hbm.at[pl.ds((NSUB * s + r) * EB, EB)], rowB)

                def _deg_fire(i, _):
                    pltpu.async_copy(ov, deg_sh.at[rowB.at[i]], esem, add=True)
                    return _
                lax.fori_loop(0, EB, _deg_fire, None)

                def _deg_drain(i, _):
                    pltpu.make_async_copy(ov, deg_sh.at[rowB.at[0]], esem).wait()
                    return _
                lax.fori_loop(0, EB, _deg_drain, None)
        plsc.subcore_barrier()

        with jax.named_scope("dis"):
            doff = s * DEG_SL
            pltpu.sync_copy(deg_sh.at[pl.ds(doff, DEG_SL)],
                            disv.at[pl.ds(0, DEG_SL)])

            def _dis(i, _):
                sl = pl.ds(i * 16, 16)
                disv[sl] = _rsqrt16(disv[sl])
                return _
            lax.fori_loop(0, DEG_SL // 16, _dis, None)
            pltpu.sync_copy(disv.at[pl.ds(0, DEG_SL)],
                            deg_sh.at[pl.ds(doff, DEG_SL)])
            plsc.subcore_barrier()
            pltpu.sync_copy(deg_sh, disv)

        def _combo(i, _):
            hf = i
            srow = (c * NS + s) * EB

            @pl.when(hf == 0)
            def _():
                with jax.named_scope("slab"):
                    pltpu.sync_copy(row_hbm.at[pl.ds(srow, EB)], rowB)
                    pltpu.sync_copy(col_hbm.at[pl.ds(srow, EB)], colB)
                    pltpu.sync_copy(cns_hbm.at[pl.ds(srow, EB)], cnsB)

                    def _norm(ii, _2):
                        for g in range(K // 16):
                            sl = pl.ds(g * 16, 16)
                            dr = plsc.load_gather(disv, [rowB[ii, sl]])
                            dc = plsc.load_gather(disv, [colB[ii, sl]])
                            cnsB[ii, sl] = dr * dc * jnp.exp(cnsB[ii, sl])
                        return _2
                    lax.fori_loop(0, EB, _norm, None)

            @pl.when(hf == 1)
            def _():
                def _shift(ii, _2):
                    for g in range(K // 16):
                        sl = pl.ds(g * 16, 16)
                        rowB[ii, sl] = rowB[ii, sl] + N_NODES
                    return _2
                lax.fori_loop(0, EB, _shift, None)

            with jax.named_scope("pipe"):
                def _sbuf_zero(e, _2):
                    for j in range(DH // 16):
                        z = jnp.zeros((16,), jnp.float32)
                        sb0[e, pl.ds(j * 16, 16)] = z
                        sb1[e, pl.ds(j * 16, 16)] = z
                        sb2[e, pl.ds(j * 16, 16)] = z
                    return _2
                lax.fori_loop(0, K, _sbuf_zero, None)
                for t in range(NSR):
                    pltpu.async_copy(sbufs[t], out_sh.at[colB.at[0]], ssems[t],
                                     add=True)
                for t in range(NG):
                    pltpu.async_copy(h2_hbm.at[rowB.at[t]], gbufs[t], gsems[t])

                def _oct(q, _2):
                    for t in range(NG):
                        b = NG * q + t
                        st = t % NSR
                        pltpu.make_async_copy(h2_hbm.at[rowB.at[0]], gbufs[t],
                                              gsems[t]).wait()
                        pltpu.make_async_copy(sbufs[st], out_sh.at[colB.at[0]],
                                              ssems[st]).wait()
                        gbuf, sbuf = gbufs[t], sbufs[st]
                        for g in range(K // 16):
                            nv = cnsB[b, pl.ds(g * 16, 16)]
                            r0 = g * 16
                            for e in range(16):
                                sc = nv[e]
                                for jj in range(DH // 16):
                                    sl = pl.ds(jj * 16, 16)
                                    sbuf[r0 + e, sl] = gbuf[r0 + e, sl] * sc
                        pltpu.async_copy(sbufs[st], out_sh.at[colB.at[b]],
                                         ssems[st], add=True)

                        @pl.when(b + NG < EB)
                        def _():
                            pltpu.async_copy(h2_hbm.at[rowB.at[b + NG]],
                                             gbufs[t], gsems[t])
                    return _2
                lax.fori_loop(0, EB // NG, _oct, None)

                for t in range(NSR):
                    pltpu.make_async_copy(sbufs[t], out_sh.at[colB.at[0]],
                                          ssems[t]).wait()
                plsc.subcore_barrier()

            with jax.named_scope("dump"):
                pltpu.sync_copy(out_sh.at[pl.ds(obase, DEG_SL)],
                                out_hbm.at[pl.ds((c * 2 + i) * N_PAD + obase,
                                                 DEG_SL)])

                def _gb_rezero(e, _2):
                    for j in range(DH // 16):
                        gb0[e, pl.ds(j * 16, 16)] = jnp.zeros((16,), jnp.float32)
                    return _2
                lax.fori_loop(0, K, _gb_rezero, None)
                _zero_out_sh()
                plsc.subcore_barrier()
            return _
        lax.fori_loop(0, 2, _combo, None)

    _core0()


_sc_propagate = functools.partial(
    pl.kernel,
    out_type=jax.ShapeDtypeStruct((NSUB * 2 * N_PAD, DH), jnp.float32),
    mesh=plsc.VectorSubcoreMesh(core_axis_name="c", subcore_axis_name="s"),
    compiler_params=pltpu.CompilerParams(needs_layout_passes=False,
                                         use_tc_tiling_on_sc=False),
    scratch_types=(
        [
            pltpu.VMEM((EB, K), jnp.int32),
            pltpu.VMEM((EB, K), jnp.int32),
            pltpu.VMEM((EB, K), jnp.float32),
            pltpu.VMEM((N_PAD,), jnp.float32),
            pltpu.VMEM((K,), jnp.float32),
        ]
        + [pltpu.VMEM((K, DH), jnp.float32)] * (NG + NSR)
        + [
            pltpu.VMEM_SHARED((N_PAD,), jnp.float32),
            pltpu.VMEM_SHARED((N_PAD, DH), jnp.float32),
        ]
        + [pltpu.SemaphoreType.DMA] * (1 + NG + NSR)
    ),
)(_sc_body)


def _mm_body(x_ref, w_ref, o_ref):
    o_ref[...] = lax.dot_general(
        x_ref[...], w_ref[...], (((1,), (1,)), ((), ())),
        preferred_element_type=jnp.float32)


def _comb_body(p0, p1, p2, p3, b_ref, o_ref):
    o_ref[:, :DH] = p0[0] + p2[0] + b_ref[0, :DH]
    o_ref[:, DH:] = p1[0] + p3[0] + b_ref[0, DH:]


def kernel(x, edge_index, cns, W, bias):
    n, d_in = x.shape
    d_out = W.shape[0]
    nblk = 10
    h = pl.pallas_call(
        _mm_body,
        grid=(nblk,),
        in_specs=[
            pl.BlockSpec((n // nblk, d_in), lambda i: (i, 0)),
            pl.BlockSpec((d_out, d_in), lambda i: (0, 0)),
        ],
        out_specs=pl.BlockSpec((n // nblk, d_out), lambda i: (i, 0)),
        out_shape=jax.ShapeDtypeStruct((n, d_out), jnp.float32),
    )(x, W)

    n_edges = edge_index.shape[1]
    pad = E_PAD - n_edges
    row2 = jnp.concatenate(
        [edge_index[0], jnp.zeros((pad,), jnp.int32)]).reshape(-1, K)
    pad_cols = N_NODES + (jnp.arange(pad, dtype=jnp.int32) % (N_PAD - N_NODES))
    col2 = jnp.concatenate([edge_index[1], pad_cols]).reshape(-1, K)
    cns2 = jnp.concatenate(
        [cns, jnp.full((pad,), -1e4, cns.dtype)]).reshape(-1, K)
    h2 = jnp.concatenate([h[:, :DH], h[:, DH:]], axis=0)
    part = _sc_propagate(h2, row2, col2, cns2).reshape(NSUB * 2, N_PAD, DH)

    out = pl.pallas_call(
        _comb_body,
        grid=(nblk,),
        in_specs=[
            pl.BlockSpec((1, n // nblk, DH), lambda i: (0, i, 0)),
            pl.BlockSpec((1, n // nblk, DH), lambda i: (1, i, 0)),
            pl.BlockSpec((1, n // nblk, DH), lambda i: (2, i, 0)),
            pl.BlockSpec((1, n // nblk, DH), lambda i: (3, i, 0)),
            pl.BlockSpec((1, d_out), lambda i: (0, 0)),
        ],
        out_specs=pl.BlockSpec((n // nblk, d_out), lambda i: (i, 0)),
        out_shape=jax.ShapeDtypeStruct((n, d_out), jnp.float32),
    )(part, part, part, part, bias.reshape(1, d_out))
    return out
